# baseline (speedup 1.0000x reference)
import functools

import jax
import jax.numpy as jnp
from jax import lax
from jax.experimental import pallas as pl
from jax.experimental.pallas import tpu as pltpu
from jax.experimental.pallas import tpu_sc as plsc

NC = 2
NS = 16
NW = NC * NS
K = 80


def _mesh():
    return plsc.VectorSubcoreMesh(core_axis_name="c", subcore_axis_name="s")


def _make_deg(N, E):
    EPW = E // NW
    NCH = EPW // K
    G = 25
    NG = NCH // G

    @functools.partial(
        pl.kernel,
        out_type=jax.ShapeDtypeStruct((NC, N, 16), jnp.float32),
        mesh=_mesh(),
        compiler_params=pltpu.CompilerParams(use_tc_tiling_on_sc=False),
        scratch_types=[
            pltpu.VMEM((NCH, K), jnp.int32),
            pltpu.VMEM((K, 16), jnp.float32),
            pltpu.VMEM_SHARED((N, 16), jnp.float32),
            pltpu.SemaphoreType.DMA,
        ],
    )
    def deg_kernel(dst_hbm, ones_hbm, zeros_hbm, out_hbm, idx_v, ones_v, accum, sem):
        c = lax.axis_index("c")
        s = lax.axis_index("s")
        w = s * NC + c
        rp = N // NS
        pltpu.sync_copy(dst_hbm.at[w], idx_v)
        pltpu.sync_copy(ones_hbm, ones_v)
        pltpu.sync_copy(zeros_hbm.at[pl.ds(s * rp, rp)], accum.at[pl.ds(s * rp, rp)])
        plsc.subcore_barrier()

        def body(g, carry):
            base = g * G
            for t in range(G):
                pltpu.async_copy(ones_v, accum.at[idx_v.at[base + t]], sem, add=True)
            for t in range(G):
                pltpu.make_async_copy(ones_v, accum.at[idx_v.at[base]], sem).wait()
            return carry

        lax.fori_loop(0, NG, body, 0)
        plsc.subcore_barrier()
        pltpu.sync_copy(accum.at[pl.ds(s * rp, rp)], out_hbm.at[c, pl.ds(s * rp, rp)])

    return deg_kernel


def _make_agg(N, E, D, NBUF, KA):
    EPW = E // NW
    NCH = EPW // KA
    NGRP = NCH // NBUF

    @functools.partial(
        pl.kernel,
        out_type=jax.ShapeDtypeStruct((NC, N, D), jnp.float32),
        mesh=_mesh(),
        compiler_params=pltpu.CompilerParams(use_tc_tiling_on_sc=False),
        scratch_types=(
            [pltpu.VMEM((NCH, KA), jnp.int32)] * 2
            + [pltpu.VMEM((KA, D), jnp.float32)] * NBUF
            + [pltpu.SemaphoreType.DMA] * (2 * NBUF)
            + [pltpu.VMEM_SHARED((N, D), jnp.float32)]
        ),
    )
    def agg_kernel(table, srci, dsti, zeros, out, sidx, didx, *rest):
        rows = rest[0:NBUF]
        gsem = rest[NBUF:2 * NBUF]
        ssem = rest[2 * NBUF:3 * NBUF]
        accum = rest[3 * NBUF]
        c = lax.axis_index("c")
        s = lax.axis_index("s")
        w = s * NC + c
        rp = N // NS
        pltpu.sync_copy(srci.at[w], sidx)
        pltpu.sync_copy(dsti.at[w], didx)
        pltpu.sync_copy(zeros.at[pl.ds(s * rp, rp)], accum.at[pl.ds(s * rp, rp)])
        plsc.subcore_barrier()

        for b in range(NBUF):
            pltpu.async_copy(table.at[sidx.at[b]], rows[b], gsem[b])

        def body(g, carry):
            for b in range(NBUF):
                j = g * NBUF + b
                pltpu.make_async_copy(table.at[sidx.at[j]], rows[b], gsem[b]).wait()
                pltpu.async_copy(rows[b], accum.at[didx.at[j]], ssem[b], add=True)
            for b in range(NBUF):
                jn = g * NBUF + NBUF + b

                @pl.when(jn < NCH)
                def _():
                    pltpu.make_async_copy(rows[b], accum.at[didx.at[0]], ssem[b]).wait()
                    pltpu.async_copy(table.at[sidx.at[jn]], rows[b], gsem[b])

            return carry

        lax.fori_loop(0, NGRP, body, 0)
        for j in range(NGRP * NBUF, NCH):
            b = j % NBUF
            pltpu.make_async_copy(table.at[sidx.at[j]], rows[b], gsem[b]).wait()
            pltpu.async_copy(rows[b], accum.at[didx.at[j]], ssem[b], add=True)
        for b in range(NBUF):
            pltpu.make_async_copy(rows[b], accum.at[didx.at[0]], ssem[b]).wait()
        plsc.subcore_barrier()
        pltpu.sync_copy(accum.at[pl.ds(s * rp, rp)], out.at[c, pl.ds(s * rp, rp)])

    return agg_kernel


def _make_decode(N, E, D):
    EPW = E // NW
    NCH = EPW // K
    NT2 = D // 32
    DB = 4
    RP = N // NS
    RC = 125
    NRC = RP // RC

    @functools.partial(
        pl.kernel,
        out_type=(jax.ShapeDtypeStruct((E,), jnp.float32),
                  jax.ShapeDtypeStruct((NC * N, D), jnp.bfloat16)),
        mesh=_mesh(),
        compiler_params=pltpu.CompilerParams(
            use_tc_tiling_on_sc=False, needs_layout_passes=False),
        scratch_types=(
            [pltpu.VMEM((NCH, K), jnp.int32)] * 2
            + [pltpu.VMEM((K, D), jnp.bfloat16)] * (2 * DB)
            + [pltpu.VMEM((K,), jnp.float32)]
            + [pltpu.SemaphoreType.DMA] * (2 * DB)
            + [pltpu.VMEM((RC, D), jnp.float32)] * 3
            + [pltpu.VMEM((RC, 16), jnp.float32)]
            + [pltpu.VMEM((1, D), jnp.float32)]
            + [pltpu.VMEM((RC, D), jnp.bfloat16)]
        ),
    )
    def dec_kernel(q, gp, dinv, b2, rowi, coli, out, zt, ridx, cidx, *rest):
        bR = rest[0:DB]
        bC = rest[DB:2 * DB]
        sv = rest[2 * DB]
        sR = rest[2 * DB + 1:3 * DB + 1]
        sC = rest[3 * DB + 1:4 * DB + 1]
        q0v, q1v, gpv = rest[4 * DB + 1:4 * DB + 4]
        dv = rest[4 * DB + 4]
        b2v = rest[4 * DB + 5]
        zb = rest[4 * DB + 6]
        c = lax.axis_index("c")
        s = lax.axis_index("s")
        w = s * NC + c

        pltpu.sync_copy(b2, b2v)

        def stage(t5, carry):
            base = s * RP + t5 * RC
            pltpu.sync_copy(q.at[0, pl.ds(base, RC)], q0v)
            pltpu.sync_copy(q.at[1, pl.ds(base, RC)], q1v)
            pltpu.sync_copy(gp.at[pl.ds(base, RC)], gpv)
            pltpu.sync_copy(dinv.at[pl.ds(base, RC)], dv)

            @plsc.parallel_loop(0, RC)
            def row(r):
                dd = dv[r, pl.ds(0, 16)]
                for tt in range(NT2):
                    v0 = (q0v[r, pl.ds(tt * 32, 16)] + q1v[r, pl.ds(tt * 32, 16)]
                          + gpv[r, pl.ds(tt * 32, 16)]) * dd + b2v[0, pl.ds(tt * 32, 16)]
                    v1 = (q0v[r, pl.ds(tt * 32 + 16, 16)] + q1v[r, pl.ds(tt * 32 + 16, 16)]
                          + gpv[r, pl.ds(tt * 32 + 16, 16)]) * dd + b2v[0, pl.ds(tt * 32 + 16, 16)]
                    zb[r, pl.ds(tt * 32, 32)] = plsc.pack(
                        v0, v1, format=plsc.PackFormat.INTERLEAVED)

            pltpu.sync_copy(zb, zt.at[pl.ds(c * N + base, RC)])
            return carry

        lax.fori_loop(0, NRC, stage, 0)
        plsc.subcore_barrier()

        pltpu.sync_copy(rowi.at[w], ridx)
        pltpu.sync_copy(coli.at[w], cidx)
        off = c * N

        @plsc.parallel_loop(0, NCH)
        def fix(ch):
            for g in range(K // 16):
                ridx[ch, pl.ds(g * 16, 16)] = ridx[ch, pl.ds(g * 16, 16)] + off
                cidx[ch, pl.ds(g * 16, 16)] = cidx[ch, pl.ds(g * 16, 16)] + off

        def gath(j, b):
            pltpu.async_copy(zt.at[ridx.at[j]], bR[b], sR[b])
            pltpu.async_copy(zt.at[cidx.at[j]], bC[b], sC[b])

        def waitg(b):
            pltpu.make_async_copy(zt.at[ridx.at[0]], bR[b], sR[b]).wait()
            pltpu.make_async_copy(zt.at[cidx.at[0]], bC[b], sC[b]).wait()

        def compute(j, bR, bC):
            lane = lax.iota(jnp.int32, 16)
            m0 = lane == 0
            perms = [lane ^ b for b in (8, 4, 2, 1)]
            gdn = lax.GatherDimensionNumbers(
                offset_dims=(), collapsed_slice_dims=(0,), start_index_map=(0,))

            def xlane(v, pv):
                return lax.gather(
                    v, pv[:, None], dimension_numbers=gdn, slice_sizes=(1,),
                    mode=lax.GatherScatterMode.PROMISE_IN_BOUNDS)

            @plsc.parallel_loop(0, K)
            def edge(e):
                acc = jnp.zeros((16,), jnp.float32)
                for t in range(NT2):
                    p = bR[e, pl.ds(t * 32, 32)] * bC[e, pl.ds(t * 32, 32)]
                    p0, p1 = plsc.unpack(p, format=plsc.PackFormat.INTERLEAVED)
                    acc = acc + p0 + p1
                for pv in perms:
                    acc = acc + xlane(acc, pv)
                plsc.store_scatter(sv, [jnp.full((16,), e, jnp.int32)], acc, mask=m0)

            @plsc.parallel_loop(0, K // 16)
            def grp(gi):
                v = sv[pl.ds(gi * 16, 16)]
                sv[pl.ds(gi * 16, 16)] = 1.0 / (1.0 + jnp.exp(-v))
            pltpu.sync_copy(sv, out.at[pl.ds(w * EPW + j * K, K)])

        for b in range(DB):
            gath(b, b)

        def body(g, carry):
            for b in range(DB):
                j = g * DB + b
                waitg(b)
                compute(j, bR[b], bC[b])
                jn = j + DB

                @pl.when(jn < NCH)
                def _():
                    gath(jn, b)

            return carry

        lax.fori_loop(0, NCH // DB, body, 0)
        for j in range((NCH // DB) * DB, NCH):
            b = j % DB
            waitg(b)
            compute(j, bR[b], bC[b])

    return dec_kernel


def _tc1_body(x_ref, w_ref, degp_ref, hp_ref, dinv_ref):
    d = degp_ref[...]
    deg = d[0, :, :1] + d[1, :, :1] + 1.0
    dinv = lax.rsqrt(deg)
    h = jnp.dot(x_ref[...], w_ref[...], preferred_element_type=jnp.float32)
    hp_ref[...] = h * dinv
    dinv_ref[...] = jnp.broadcast_to(dinv, (dinv.shape[0], 16))


def _tc2_body(p_ref, hp_ref, dinv_ref, b1_ref, w2_ref, gp_ref):
    p = p_ref[...]
    dinv = dinv_ref[...][:, :1]
    t = (p[0] + p[1] + hp_ref[...]) * dinv + b1_ref[...]
    r = jnp.maximum(t, 0.0)
    g = jnp.dot(r, w2_ref[...], preferred_element_type=jnp.float32)
    gp_ref[...] = g * dinv


def _tc3_body(q_ref, gp_ref, dinv_ref, b2_ref, z_ref):
    q = q_ref[...]
    z = (q[0] + q[1] + gp_ref[...]) * dinv_ref[...] + b2_ref[...]
    z_ref[...] = z.astype(jnp.bfloat16)


_BLK = 1000


def _tc1(x, W1, degp):
    N, Din = x.shape
    Dh = W1.shape[1]
    return pl.pallas_call(
        _tc1_body,
        grid=(N // _BLK,),
        in_specs=[
            pl.BlockSpec((_BLK, Din), lambda i: (i, 0)),
            pl.BlockSpec((Din, Dh), lambda i: (0, 0)),
            pl.BlockSpec((NC, _BLK, 16), lambda i: (0, i, 0)),
        ],
        out_specs=[
            pl.BlockSpec((_BLK, Dh), lambda i: (i, 0)),
            pl.BlockSpec((_BLK, 16), lambda i: (i, 0)),
        ],
        out_shape=[
            jax.ShapeDtypeStruct((N, Dh), jnp.float32),
            jax.ShapeDtypeStruct((N, 16), jnp.float32),
        ],
    )(x, W1, degp)


def _tc2(p, hp, dinv, b1, W2):
    N, Dh = hp.shape
    Dl = W2.shape[1]
    return pl.pallas_call(
        _tc2_body,
        grid=(N // _BLK,),
        in_specs=[
            pl.BlockSpec((NC, _BLK, Dh), lambda i: (0, i, 0)),
            pl.BlockSpec((_BLK, Dh), lambda i: (i, 0)),
            pl.BlockSpec((_BLK, 16), lambda i: (i, 0)),
            pl.BlockSpec((1, Dh), lambda i: (0, 0)),
            pl.BlockSpec((Dh, Dl), lambda i: (0, 0)),
        ],
        out_specs=pl.BlockSpec((_BLK, Dl), lambda i: (i, 0)),
        out_shape=jax.ShapeDtypeStruct((N, Dl), jnp.float32),
    )(p, hp, dinv, b1, W2)


def _tc3(q, gp, dinv, b2):
    N, Dl = gp.shape
    return pl.pallas_call(
        _tc3_body,
        grid=(N // _BLK,),
        in_specs=[
            pl.BlockSpec((NC, _BLK, Dl), lambda i: (0, i, 0)),
            pl.BlockSpec((_BLK, Dl), lambda i: (i, 0)),
            pl.BlockSpec((_BLK, 1), lambda i: (i, 0)),
            pl.BlockSpec((1, Dl), lambda i: (0, 0)),
        ],
        out_specs=pl.BlockSpec((_BLK, Dl), lambda i: (i, 0)),
        out_shape=jax.ShapeDtypeStruct((N, Dl), jnp.bfloat16),
    )(q, gp, dinv, b2)


def kernel(x, edge_index, W1, b1, W2, b2):
    N, Din = x.shape
    Dh = W1.shape[1]
    Dl = W2.shape[1]
    E = edge_index.shape[1]
    EPW = E // NW
    NCH = EPW // K
    KA = K
    src = edge_index[0].reshape(NW, NCH, K)
    dst = edge_index[1].reshape(NW, NCH, K)
    srca = edge_index[0].reshape(NW, EPW // KA, KA)
    dsta = edge_index[1].reshape(NW, EPW // KA, KA)
    ones16 = jnp.ones((K, 16), jnp.float32)
    z16 = jnp.zeros((N, 16), jnp.float32)
    zh = jnp.zeros((N, Dh), jnp.float32)
    zl = jnp.zeros((N, Dl), jnp.float32)

    degp = _make_deg(N, E)(dst, ones16, z16)
    hp, dinv = _tc1(x, W1, degp)
    p = _make_agg(N, E, Dh, 3, KA)(hp, srca, dsta, zh)
    gp = _tc2(p, hp, dinv, b1.reshape(1, Dh), W2)
    q = _make_agg(N, E, Dl, 12, KA)(gp, srca, dsta, zl)
    scores, _ = _make_decode(N, E, Dl)(q, gp, dinv, b2.reshape(1, Dl), src, dst)
    return scores


# speedup vs baseline: 1.0046x; 1.0046x over previous
import functools

import jax
import jax.numpy as jnp
from jax import lax
from jax.experimental import pallas as pl
from jax.experimental.pallas import tpu as pltpu
from jax.experimental.pallas import tpu_sc as plsc

NC = 2
NS = 16
NW = NC * NS
K = 80


def _mesh():
    return plsc.VectorSubcoreMesh(core_axis_name="c", subcore_axis_name="s")


def _make_deg(N, E):
    EPW = E // NW
    NCH = EPW // K
    G = 25
    NG = NCH // G

    @functools.partial(
        pl.kernel,
        out_type=jax.ShapeDtypeStruct((NC, N, 16), jnp.float32),
        mesh=_mesh(),
        compiler_params=pltpu.CompilerParams(use_tc_tiling_on_sc=False),
        scratch_types=[
            pltpu.VMEM((NCH, K), jnp.int32),
            pltpu.VMEM((K, 16), jnp.float32),
            pltpu.VMEM_SHARED((N, 16), jnp.float32),
            pltpu.SemaphoreType.DMA,
        ],
    )
    def deg_kernel(dst_hbm, ones_hbm, zeros_hbm, out_hbm, idx_v, ones_v, accum, sem):
        c = lax.axis_index("c")
        s = lax.axis_index("s")
        w = s * NC + c
        rp = N // NS
        pltpu.sync_copy(dst_hbm.at[w], idx_v)
        pltpu.sync_copy(ones_hbm, ones_v)
        pltpu.sync_copy(zeros_hbm.at[pl.ds(s * rp, rp)], accum.at[pl.ds(s * rp, rp)])
        plsc.subcore_barrier()

        def body(g, carry):
            base = g * G
            for t in range(G):
                pltpu.async_copy(ones_v, accum.at[idx_v.at[base + t]], sem, add=True)
            for t in range(G):
                pltpu.make_async_copy(ones_v, accum.at[idx_v.at[base]], sem).wait()
            return carry

        lax.fori_loop(0, NG, body, 0)
        plsc.subcore_barrier()
        pltpu.sync_copy(accum.at[pl.ds(s * rp, rp)], out_hbm.at[c, pl.ds(s * rp, rp)])

    return deg_kernel


def _make_agg(N, E, D, NBUF, KA):
    EPW = E // NW
    NCH = EPW // KA
    NGRP = NCH // NBUF

    @functools.partial(
        pl.kernel,
        out_type=jax.ShapeDtypeStruct((NC, N, D), jnp.float32),
        mesh=_mesh(),
        compiler_params=pltpu.CompilerParams(use_tc_tiling_on_sc=False),
        scratch_types=(
            [pltpu.VMEM((NCH, KA), jnp.int32)] * 2
            + [pltpu.VMEM((KA, D), jnp.float32)] * NBUF
            + [pltpu.SemaphoreType.DMA] * (2 * NBUF)
            + [pltpu.VMEM_SHARED((N, D), jnp.float32)]
        ),
    )
    def agg_kernel(table, srci, dsti, zeros, out, sidx, didx, *rest):
        rows = rest[0:NBUF]
        gsem = rest[NBUF:2 * NBUF]
        ssem = rest[2 * NBUF:3 * NBUF]
        accum = rest[3 * NBUF]
        c = lax.axis_index("c")
        s = lax.axis_index("s")
        w = s * NC + c
        rp = N // NS
        pltpu.sync_copy(srci.at[w], sidx)
        pltpu.sync_copy(dsti.at[w], didx)
        pltpu.sync_copy(zeros.at[pl.ds(s * rp, rp)], accum.at[pl.ds(s * rp, rp)])
        plsc.subcore_barrier()

        for b in range(NBUF):
            pltpu.async_copy(table.at[sidx.at[b]], rows[b], gsem[b])

        def body(g, carry):
            for b in range(NBUF):
                j = g * NBUF + b
                pltpu.make_async_copy(table.at[sidx.at[j]], rows[b], gsem[b]).wait()
                pltpu.async_copy(rows[b], accum.at[didx.at[j]], ssem[b], add=True)
            for b in range(NBUF):
                jn = g * NBUF + NBUF + b

                @pl.when(jn < NCH)
                def _():
                    pltpu.make_async_copy(rows[b], accum.at[didx.at[0]], ssem[b]).wait()
                    pltpu.async_copy(table.at[sidx.at[jn]], rows[b], gsem[b])

            return carry

        lax.fori_loop(0, NGRP, body, 0)
        for j in range(NGRP * NBUF, NCH):
            b = j % NBUF
            pltpu.make_async_copy(table.at[sidx.at[j]], rows[b], gsem[b]).wait()
            pltpu.async_copy(rows[b], accum.at[didx.at[j]], ssem[b], add=True)
        for b in range(NBUF):
            pltpu.make_async_copy(rows[b], accum.at[didx.at[0]], ssem[b]).wait()
        plsc.subcore_barrier()
        pltpu.sync_copy(accum.at[pl.ds(s * rp, rp)], out.at[c, pl.ds(s * rp, rp)])

    return agg_kernel


def _make_decode(N, E, D):
    EPW = E // NW
    NCH = EPW // K
    NT2 = D // 32
    DB = 6

    @functools.partial(
        pl.kernel,
        out_type=jax.ShapeDtypeStruct((E,), jnp.float32),
        mesh=_mesh(),
        compiler_params=pltpu.CompilerParams(
            use_tc_tiling_on_sc=False, needs_layout_passes=False),
        scratch_types=(
            [pltpu.VMEM((NCH, K), jnp.int32)] * 2
            + [pltpu.VMEM((K, D), jnp.bfloat16)] * (2 * DB)
            + [pltpu.VMEM((K,), jnp.float32)]
            + [pltpu.SemaphoreType.DMA] * (2 * DB)
        ),
    )
    def dec_kernel(z, rowi, coli, out, ridx, cidx, *rest):
        bR = rest[0:DB]
        bC = rest[DB:2 * DB]
        sv = rest[2 * DB]
        sR = rest[2 * DB + 1:3 * DB + 1]
        sC = rest[3 * DB + 1:4 * DB + 1]
        c = lax.axis_index("c")
        s = lax.axis_index("s")
        w = s * NC + c
        pltpu.sync_copy(rowi.at[w], ridx)
        pltpu.sync_copy(coli.at[w], cidx)

        def gath(j, b):
            pltpu.async_copy(z.at[ridx.at[j]], bR[b], sR[b])
            pltpu.async_copy(z.at[cidx.at[j]], bC[b], sC[b])

        def waitg(b):
            pltpu.make_async_copy(z.at[ridx.at[0]], bR[b], sR[b]).wait()
            pltpu.make_async_copy(z.at[cidx.at[0]], bC[b], sC[b]).wait()

        def compute(j, bR, bC):
            lane = lax.iota(jnp.int32, 16)
            m0 = lane == 0
            perms = [lane ^ b for b in (8, 4, 2, 1)]
            gdn = lax.GatherDimensionNumbers(
                offset_dims=(), collapsed_slice_dims=(0,), start_index_map=(0,))

            def xlane(v, pv):
                return lax.gather(
                    v, pv[:, None], dimension_numbers=gdn, slice_sizes=(1,),
                    mode=lax.GatherScatterMode.PROMISE_IN_BOUNDS)

            @plsc.parallel_loop(0, K)
            def edge(e):
                acc = jnp.zeros((16,), jnp.float32)
                for t in range(NT2):
                    p = bR[e, pl.ds(t * 32, 32)] * bC[e, pl.ds(t * 32, 32)]
                    p0, p1 = plsc.unpack(p, format=plsc.PackFormat.INTERLEAVED)
                    acc = acc + p0 + p1
                for pv in perms:
                    acc = acc + xlane(acc, pv)
                plsc.store_scatter(sv, [jnp.full((16,), e, jnp.int32)], acc, mask=m0)

            @plsc.parallel_loop(0, K // 16)
            def grp(gi):
                v = sv[pl.ds(gi * 16, 16)]
                sv[pl.ds(gi * 16, 16)] = 1.0 / (1.0 + jnp.exp(-v))
            pltpu.sync_copy(sv, out.at[pl.ds(w * EPW + j * K, K)])

        for b in range(DB):
            gath(b, b)

        def body(g, carry):
            for b in range(DB):
                j = g * DB + b
                waitg(b)
                compute(j, bR[b], bC[b])
                jn = j + DB

                @pl.when(jn < NCH)
                def _():
                    gath(jn, b)

            return carry

        lax.fori_loop(0, NCH // DB, body, 0)
        for j in range((NCH // DB) * DB, NCH):
            b = j % DB
            waitg(b)
            compute(j, bR[b], bC[b])

    return dec_kernel


def _tc1_body(x_ref, w_ref, degp_ref, hp_ref, dinv_ref):
    d = degp_ref[...]
    deg = d[0, :, :1] + d[1, :, :1] + 1.0
    dinv = lax.rsqrt(deg)
    h = jnp.dot(x_ref[...], w_ref[...], preferred_element_type=jnp.float32)
    hp_ref[...] = h * dinv
    dinv_ref[...] = dinv


def _tc2_body(p_ref, hp_ref, dinv_ref, b1_ref, w2_ref, gp_ref):
    p = p_ref[...]
    t = (p[0] + p[1] + hp_ref[...]) * dinv_ref[...] + b1_ref[...]
    r = jnp.maximum(t, 0.0)
    g = jnp.dot(r, w2_ref[...], preferred_element_type=jnp.float32)
    gp_ref[...] = g * dinv_ref[...]


def _tc3_body(q_ref, gp_ref, dinv_ref, b2_ref, z_ref):
    q = q_ref[...]
    z = (q[0] + q[1] + gp_ref[...]) * dinv_ref[...] + b2_ref[...]
    z_ref[...] = z.astype(jnp.bfloat16)


_BLK = 1000


def _tc1(x, W1, degp):
    N, Din = x.shape
    Dh = W1.shape[1]
    return pl.pallas_call(
        _tc1_body,
        grid=(N // _BLK,),
        in_specs=[
            pl.BlockSpec((_BLK, Din), lambda i: (i, 0)),
            pl.BlockSpec((Din, Dh), lambda i: (0, 0)),
            pl.BlockSpec((NC, _BLK, 16), lambda i: (0, i, 0)),
        ],
        out_specs=[
            pl.BlockSpec((_BLK, Dh), lambda i: (i, 0)),
            pl.BlockSpec((_BLK, 1), lambda i: (i, 0)),
        ],
        out_shape=[
            jax.ShapeDtypeStruct((N, Dh), jnp.float32),
            jax.ShapeDtypeStruct((N, 1), jnp.float32),
        ],
    )(x, W1, degp)


def _tc2(p, hp, dinv, b1, W2):
    N, Dh = hp.shape
    Dl = W2.shape[1]
    return pl.pallas_call(
        _tc2_body,
        grid=(N // _BLK,),
        in_specs=[
            pl.BlockSpec((NC, _BLK, Dh), lambda i: (0, i, 0)),
            pl.BlockSpec((_BLK, Dh), lambda i: (i, 0)),
            pl.BlockSpec((_BLK, 1), lambda i: (i, 0)),
            pl.BlockSpec((1, Dh), lambda i: (0, 0)),
            pl.BlockSpec((Dh, Dl), lambda i: (0, 0)),
        ],
        out_specs=pl.BlockSpec((_BLK, Dl), lambda i: (i, 0)),
        out_shape=jax.ShapeDtypeStruct((N, Dl), jnp.float32),
    )(p, hp, dinv, b1, W2)


def _tc3(q, gp, dinv, b2):
    N, Dl = gp.shape
    return pl.pallas_call(
        _tc3_body,
        grid=(N // _BLK,),
        in_specs=[
            pl.BlockSpec((NC, _BLK, Dl), lambda i: (0, i, 0)),
            pl.BlockSpec((_BLK, Dl), lambda i: (i, 0)),
            pl.BlockSpec((_BLK, 1), lambda i: (i, 0)),
            pl.BlockSpec((1, Dl), lambda i: (0, 0)),
        ],
        out_specs=pl.BlockSpec((_BLK, Dl), lambda i: (i, 0)),
        out_shape=jax.ShapeDtypeStruct((N, Dl), jnp.bfloat16),
    )(q, gp, dinv, b2)


def kernel(x, edge_index, W1, b1, W2, b2):
    N, Din = x.shape
    Dh = W1.shape[1]
    Dl = W2.shape[1]
    E = edge_index.shape[1]
    EPW = E // NW
    NCH = EPW // K
    KA = K
    src = edge_index[0].reshape(NW, NCH, K)
    dst = edge_index[1].reshape(NW, NCH, K)
    srca = edge_index[0].reshape(NW, EPW // KA, KA)
    dsta = edge_index[1].reshape(NW, EPW // KA, KA)
    ones16 = jnp.ones((K, 16), jnp.float32)
    z16 = jnp.zeros((N, 16), jnp.float32)
    zh = jnp.zeros((N, Dh), jnp.float32)
    zl = jnp.zeros((N, Dl), jnp.float32)

    degp = _make_deg(N, E)(dst, ones16, z16)
    hp, dinv = _tc1(x, W1, degp)
    p = _make_agg(N, E, Dh, 3, KA)(hp, srca, dsta, zh)
    gp = _tc2(p, hp, dinv, b1.reshape(1, Dh), W2)
    q = _make_agg(N, E, Dl, 12, KA)(gp, srca, dsta, zl)
    z = _tc3(q, gp, dinv, b2.reshape(1, Dl))
    return _make_decode(N, E, Dl)(z, src, dst)


# speedup vs baseline: 1.0108x; 1.0063x over previous
import functools

import jax
import jax.numpy as jnp
from jax import lax
from jax.experimental import pallas as pl
from jax.experimental.pallas import tpu as pltpu
from jax.experimental.pallas import tpu_sc as plsc

NC = 2
NS = 16
NW = NC * NS
K = 80


def _mesh():
    return plsc.VectorSubcoreMesh(core_axis_name="c", subcore_axis_name="s")


def _make_deg(N, E):
    EPW = E // NW
    NCH = EPW // K
    G = 25
    NG = NCH // G

    @functools.partial(
        pl.kernel,
        out_type=jax.ShapeDtypeStruct((NC, N, 16), jnp.float32),
        mesh=_mesh(),
        compiler_params=pltpu.CompilerParams(use_tc_tiling_on_sc=False),
        scratch_types=[
            pltpu.VMEM((NCH, K), jnp.int32),
            pltpu.VMEM((K, 16), jnp.float32),
            pltpu.VMEM_SHARED((N, 16), jnp.float32),
            pltpu.SemaphoreType.DMA,
        ],
    )
    def deg_kernel(dst_hbm, ones_hbm, zeros_hbm, out_hbm, idx_v, ones_v, accum, sem):
        c = lax.axis_index("c")
        s = lax.axis_index("s")
        w = s * NC + c
        rp = N // NS
        pltpu.sync_copy(dst_hbm.at[w], idx_v)
        pltpu.sync_copy(ones_hbm, ones_v)
        pltpu.sync_copy(zeros_hbm.at[pl.ds(s * rp, rp)], accum.at[pl.ds(s * rp, rp)])
        plsc.subcore_barrier()

        def body(g, carry):
            base = g * G
            for t in range(G):
                pltpu.async_copy(ones_v, accum.at[idx_v.at[base + t]], sem, add=True)
            for t in range(G):
                pltpu.make_async_copy(ones_v, accum.at[idx_v.at[base]], sem).wait()
            return carry

        lax.fori_loop(0, NG, body, 0)
        plsc.subcore_barrier()
        pltpu.sync_copy(accum.at[pl.ds(s * rp, rp)], out_hbm.at[c, pl.ds(s * rp, rp)])

    return deg_kernel


def _make_agg(N, E, D, NBUF, KA):
    EPW = E // NW
    NCH = EPW // KA
    NGRP = NCH // NBUF

    @functools.partial(
        pl.kernel,
        out_type=jax.ShapeDtypeStruct((NC, N, D), jnp.float32),
        mesh=_mesh(),
        compiler_params=pltpu.CompilerParams(use_tc_tiling_on_sc=False),
        scratch_types=(
            [pltpu.VMEM((NCH, KA), jnp.int32)] * 2
            + [pltpu.VMEM((KA, D), jnp.float32)] * NBUF
            + [pltpu.SemaphoreType.DMA] * (2 * NBUF)
            + [pltpu.VMEM_SHARED((N, D), jnp.float32)]
        ),
    )
    def agg_kernel(table, srci, dsti, zeros, out, sidx, didx, *rest):
        rows = rest[0:NBUF]
        gsem = rest[NBUF:2 * NBUF]
        ssem = rest[2 * NBUF:3 * NBUF]
        accum = rest[3 * NBUF]
        c = lax.axis_index("c")
        s = lax.axis_index("s")
        w = s * NC + c
        rp = N // NS
        pltpu.sync_copy(srci.at[w], sidx)
        pltpu.sync_copy(dsti.at[w], didx)
        pltpu.sync_copy(zeros.at[pl.ds(s * rp, rp)], accum.at[pl.ds(s * rp, rp)])
        plsc.subcore_barrier()

        for b in range(NBUF):
            pltpu.async_copy(table.at[sidx.at[b]], rows[b], gsem[b])

        def body(g, carry):
            for b in range(NBUF):
                j = g * NBUF + b
                pltpu.make_async_copy(table.at[sidx.at[j]], rows[b], gsem[b]).wait()
                pltpu.async_copy(rows[b], accum.at[didx.at[j]], ssem[b], add=True)
            for b in range(NBUF):
                jn = g * NBUF + NBUF + b

                @pl.when(jn < NCH)
                def _():
                    pltpu.make_async_copy(rows[b], accum.at[didx.at[0]], ssem[b]).wait()
                    pltpu.async_copy(table.at[sidx.at[jn]], rows[b], gsem[b])

            return carry

        lax.fori_loop(0, NGRP, body, 0)
        for j in range(NGRP * NBUF, NCH):
            b = j % NBUF
            pltpu.make_async_copy(table.at[sidx.at[j]], rows[b], gsem[b]).wait()
            pltpu.async_copy(rows[b], accum.at[didx.at[j]], ssem[b], add=True)
        for b in range(NBUF):
            pltpu.make_async_copy(rows[b], accum.at[didx.at[0]], ssem[b]).wait()
        plsc.subcore_barrier()
        pltpu.sync_copy(accum.at[pl.ds(s * rp, rp)], out.at[c, pl.ds(s * rp, rp)])

    return agg_kernel


def _make_decode(N, E, D):
    EPW = E // NW
    NCH = EPW // K
    NT2 = D // 32
    DB = 4

    @functools.partial(
        pl.kernel,
        out_type=jax.ShapeDtypeStruct((E,), jnp.float32),
        mesh=_mesh(),
        compiler_params=pltpu.CompilerParams(
            use_tc_tiling_on_sc=False, needs_layout_passes=False),
        scratch_types=(
            [pltpu.VMEM((NCH, K), jnp.int32)] * 2
            + [pltpu.VMEM((K, D), jnp.bfloat16)] * (2 * DB)
            + [pltpu.VMEM((K,), jnp.float32)]
            + [pltpu.SemaphoreType.DMA] * (2 * DB)
        ),
    )
    def dec_kernel(z, rowi, coli, out, ridx, cidx, *rest):
        bR = rest[0:DB]
        bC = rest[DB:2 * DB]
        sv = rest[2 * DB]
        sR = rest[2 * DB + 1:3 * DB + 1]
        sC = rest[3 * DB + 1:4 * DB + 1]
        c = lax.axis_index("c")
        s = lax.axis_index("s")
        w = s * NC + c
        pltpu.sync_copy(rowi.at[w], ridx)
        pltpu.sync_copy(coli.at[w], cidx)

        def gath(j, b):
            pltpu.async_copy(z.at[ridx.at[j]], bR[b], sR[b])
            pltpu.async_copy(z.at[cidx.at[j]], bC[b], sC[b])

        def waitg(b):
            pltpu.make_async_copy(z.at[ridx.at[0]], bR[b], sR[b]).wait()
            pltpu.make_async_copy(z.at[cidx.at[0]], bC[b], sC[b]).wait()

        def compute(j, bR, bC):
            lane = lax.iota(jnp.int32, 16)
            m0 = lane == 0
            perms = [lane ^ b for b in (8, 4, 2, 1)]
            gdn = lax.GatherDimensionNumbers(
                offset_dims=(), collapsed_slice_dims=(0,), start_index_map=(0,))

            def xlane(v, pv):
                return lax.gather(
                    v, pv[:, None], dimension_numbers=gdn, slice_sizes=(1,),
                    mode=lax.GatherScatterMode.PROMISE_IN_BOUNDS)

            @plsc.parallel_loop(0, K)
            def edge(e):
                acc = jnp.zeros((16,), jnp.float32)
                for t in range(NT2):
                    p = bR[e, pl.ds(t * 32, 32)] * bC[e, pl.ds(t * 32, 32)]
                    p0, p1 = plsc.unpack(p, format=plsc.PackFormat.INTERLEAVED)
                    acc = acc + p0 + p1
                for pv in perms:
                    acc = acc + xlane(acc, pv)
                plsc.store_scatter(sv, [jnp.full((16,), e, jnp.int32)], acc, mask=m0)

            @plsc.parallel_loop(0, K // 16)
            def grp(gi):
                v = sv[pl.ds(gi * 16, 16)]
                sv[pl.ds(gi * 16, 16)] = 1.0 / (1.0 + jnp.exp(-v))
            pltpu.sync_copy(sv, out.at[pl.ds(w * EPW + j * K, K)])

        for b in range(DB):
            gath(b, b)

        def body(g, carry):
            for b in range(DB):
                j = g * DB + b
                waitg(b)
                compute(j, bR[b], bC[b])
                jn = j + DB

                @pl.when(jn < NCH)
                def _():
                    gath(jn, b)

            return carry

        lax.fori_loop(0, NCH // DB, body, 0)
        for j in range((NCH // DB) * DB, NCH):
            b = j % DB
            waitg(b)
            compute(j, bR[b], bC[b])

    return dec_kernel


def _tc1_body(x_ref, w_ref, degp_ref, hp_ref, dinv_ref):
    d = degp_ref[...]
    deg = d[0, :, :1] + d[1, :, :1] + 1.0
    dinv = lax.rsqrt(deg)
    h = jnp.dot(x_ref[...], w_ref[...], preferred_element_type=jnp.float32)
    hp_ref[...] = h * dinv
    dinv_ref[...] = dinv


def _tc2_body(p_ref, hp_ref, dinv_ref, b1_ref, w2_ref, gp_ref):
    p = p_ref[...]
    t = (p[0] + p[1] + hp_ref[...]) * dinv_ref[...] + b1_ref[...]
    r = jnp.maximum(t, 0.0)
    g = jnp.dot(r, w2_ref[...], preferred_element_type=jnp.float32)
    gp_ref[...] = g * dinv_ref[...]


def _tc3_body(q_ref, gp_ref, dinv_ref, b2_ref, z_ref):
    q = q_ref[...]
    z = (q[0] + q[1] + gp_ref[...]) * dinv_ref[...] + b2_ref[...]
    z_ref[...] = z.astype(jnp.bfloat16)


_BLK = 1000


def _tc1(x, W1, degp):
    N, Din = x.shape
    Dh = W1.shape[1]
    return pl.pallas_call(
        _tc1_body,
        grid=(N // _BLK,),
        in_specs=[
            pl.BlockSpec((_BLK, Din), lambda i: (i, 0)),
            pl.BlockSpec((Din, Dh), lambda i: (0, 0)),
            pl.BlockSpec((NC, _BLK, 16), lambda i: (0, i, 0)),
        ],
        out_specs=[
            pl.BlockSpec((_BLK, Dh), lambda i: (i, 0)),
            pl.BlockSpec((_BLK, 1), lambda i: (i, 0)),
        ],
        out_shape=[
            jax.ShapeDtypeStruct((N, Dh), jnp.float32),
            jax.ShapeDtypeStruct((N, 1), jnp.float32),
        ],
    )(x, W1, degp)


def _tc2(p, hp, dinv, b1, W2):
    N, Dh = hp.shape
    Dl = W2.shape[1]
    return pl.pallas_call(
        _tc2_body,
        grid=(N // _BLK,),
        in_specs=[
            pl.BlockSpec((NC, _BLK, Dh), lambda i: (0, i, 0)),
            pl.BlockSpec((_BLK, Dh), lambda i: (i, 0)),
            pl.BlockSpec((_BLK, 1), lambda i: (i, 0)),
            pl.BlockSpec((1, Dh), lambda i: (0, 0)),
            pl.BlockSpec((Dh, Dl), lambda i: (0, 0)),
        ],
        out_specs=pl.BlockSpec((_BLK, Dl), lambda i: (i, 0)),
        out_shape=jax.ShapeDtypeStruct((N, Dl), jnp.float32),
    )(p, hp, dinv, b1, W2)


def _tc3(q, gp, dinv, b2):
    N, Dl = gp.shape
    return pl.pallas_call(
        _tc3_body,
        grid=(N // _BLK,),
        in_specs=[
            pl.BlockSpec((NC, _BLK, Dl), lambda i: (0, i, 0)),
            pl.BlockSpec((_BLK, Dl), lambda i: (i, 0)),
            pl.BlockSpec((_BLK, 1), lambda i: (i, 0)),
            pl.BlockSpec((1, Dl), lambda i: (0, 0)),
        ],
        out_specs=pl.BlockSpec((_BLK, Dl), lambda i: (i, 0)),
        out_shape=jax.ShapeDtypeStruct((N, Dl), jnp.bfloat16),
    )(q, gp, dinv, b2)


def kernel(x, edge_index, W1, b1, W2, b2):
    N, Din = x.shape
    Dh = W1.shape[1]
    Dl = W2.shape[1]
    E = edge_index.shape[1]
    EPW = E // NW
    NCH = EPW // K
    KA = K
    src = edge_index[0].reshape(NW, NCH, K)
    dst = edge_index[1].reshape(NW, NCH, K)
    srca = edge_index[0].reshape(NW, EPW // KA, KA)
    dsta = edge_index[1].reshape(NW, EPW // KA, KA)
    ones16 = jnp.ones((K, 16), jnp.float32)
    z16 = jnp.zeros((N, 16), jnp.float32)
    zh = jnp.zeros((N, Dh), jnp.float32)
    zl = jnp.zeros((N, Dl), jnp.float32)

    degp = _make_deg(N, E)(dst, ones16, z16)
    hp, dinv = _tc1(x, W1, degp)
    p = _make_agg(N, E, Dh, 3, KA)(hp, srca, dsta, zh)
    gp = _tc2(p, hp, dinv, b1.reshape(1, Dh), W2)
    q = _make_agg(N, E, Dl, 12, KA)(gp, srca, dsta, zl)
    z = _tc3(q, gp, dinv, b2.reshape(1, Dl))
    return _make_decode(N, E, Dl)(z, src, dst)
